# baseline (speedup 1.0000x reference)
import jax
import jax.numpy as jnp
from jax import lax
from jax.experimental import pallas as pl
from jax.experimental.pallas import tpu as pltpu
from jax.experimental.pallas import tpu_sc as plsc

_N = 4096; _D = 256; _P = 64; _H = 8; _DK = _D // _H; _A = 14
_NI = 16; _NS = 16; _K = 64
_RB = 64
_RP = 256
_RA = 64
_M = _RA * _K
_W = 640
_C0K, _C0V, _C0P, _C0R, _C0C, _C0B = 0, 256, 512, 554, 563, 564
_NW, _CH = 32, 128


_CB = 512
_NCH = _N // _CB


def _nb_body(bnd_ref, g_ref, car_ref, cat_ref, cbr_ref, cbt_ref, nb_ref,
             vals_ref):
    i = pl.program_id(0)
    i0 = i * _RB
    lo = bnd_ref[i, 0]
    hi = bnd_ref[i, 1]
    clo = lo // _CB
    chi = hi // _CB + 1
    ii = i0 + lax.broadcasted_iota(jnp.int32, (_RB, 1), 0)
    jl = lax.broadcasted_iota(jnp.int32, (1, _CB), 1)
    xi = car_ref[:, 0:1]; yi = car_ref[:, 1:2]; zi = car_ref[:, 2:3]
    ci = cbr_ref[:, 0:1]; bi = cbr_ref[:, 1:2]
    inf = jnp.float32(jnp.inf)

    def chunk_masks(c):
        jj = c * _CB + jl
        xj = cat_ref[c, 0:1, :]; yj = cat_ref[c, 1:2, :]; zj = cat_ref[c, 2:3, :]
        dx = xi - xj; dy = yi - yj; dz = zi - zj
        d2 = dx * dx + dy * dy + dz * dz
        same_b = bi == cbt_ref[c, 1:2, :]
        same_c = ci == cbt_ref[c, 0:1, :]
        within = (jnp.abs(ii - jj) < _NI) & same_b & same_c
        return d2, same_b, within

    def p1(c, _):
        d2, same_b, within = chunk_masks(c)
        vals_ref[c] = jnp.where(within | (~same_b), inf, d2)
        return 0

    lax.fori_loop(clo, chi, p1, 0)

    def cut_iter(t, _):
        def mn(c, m):
            return jnp.minimum(m, jnp.min(vals_ref[c], axis=1, keepdims=True))

        m = lax.fori_loop(clo, chi, mn, jnp.full((_RB, 1), inf, jnp.float32))

        def rm(c, _):
            v = vals_ref[c]
            vals_ref[c] = jnp.where(v == m, inf, v)
            return 0

        lax.fori_loop(clo, chi, rm, 0)
        return m

    cut2 = lax.fori_loop(0, _NS, cut_iter,
                         jnp.zeros((_RB, 1), jnp.float32))

    def p3(c, _):
        d2, same_b, within = chunk_masks(c)
        dm = jnp.where(within | (~same_b), inf, d2)
        within2 = within | (dm < cut2)
        d2e = d2 + 1e-12
        key = jnp.exp(2.0 * g_ref[c]) * (d2e * d2e * d2e)
        key = jnp.where(within2, -1.0, key)
        vals_ref[c] = jnp.where(same_b, key, inf)
        return 0

    lax.fori_loop(clo, chi, p3, 0)
    kidx = lax.broadcasted_iota(jnp.int32, (1, _K), 1)

    def ebody(t, nb):
        def mn(c, m):
            return jnp.minimum(m, jnp.min(vals_ref[c], axis=1, keepdims=True))

        m = lax.fori_loop(clo, chi, mn, jnp.full((_RB, 1), inf, jnp.float32))

        def am(c, j):
            v = vals_ref[c]
            jj = c * _CB + jl
            return jnp.minimum(
                j, jnp.min(jnp.where(v == m, jj, _N), axis=1, keepdims=True))

        j = lax.fori_loop(clo, chi, am,
                          jnp.full((_RB, 1), _N, jnp.int32))

        def rm(c, _):
            v = vals_ref[c]
            jj = c * _CB + jl
            vals_ref[c] = jnp.where(jj == j, inf, v)
            return 0

        lax.fori_loop(clo, chi, rm, 0)
        col = jnp.where(jnp.isinf(m), -1, j)
        return jnp.where(kidx == t, col, nb)

    nb = lax.fori_loop(0, _K, ebody, jnp.zeros((_RB, _K), jnp.int32))
    nb_ref[...] = nb


def _nb_pallas(ca, chain, batch, g):
    cat = ca.T.reshape(3, _NCH, _CB).transpose(1, 0, 2)
    cbr = jnp.stack([chain, batch], axis=1)
    cbt = jnp.stack([chain, batch], axis=0).reshape(2, _NCH, _CB
                                                   ).transpose(1, 0, 2)
    g3 = g.reshape(_N, _NCH, _CB).transpose(1, 0, 2)
    i0s = jnp.arange(_N // _RB, dtype=jnp.int32) * _RB
    lo = jnp.searchsorted(batch, batch[i0s], side='left').astype(jnp.int32)
    hi = (jnp.searchsorted(batch, batch[i0s + _RB - 1], side='right')
          .astype(jnp.int32) - 1)
    bounds = jnp.stack([lo, hi], axis=1)
    grid_spec = pltpu.PrefetchScalarGridSpec(
        num_scalar_prefetch=1,
        grid=(_N // _RB,),
        in_specs=[
            pl.BlockSpec((_NCH, _RB, _CB), lambda i, b: (0, i, 0)),
            pl.BlockSpec((_RB, 3), lambda i, b: (i, 0)),
            pl.BlockSpec((_NCH, 3, _CB), lambda i, b: (0, 0, 0)),
            pl.BlockSpec((_RB, 2), lambda i, b: (i, 0)),
            pl.BlockSpec((_NCH, 2, _CB), lambda i, b: (0, 0, 0)),
        ],
        out_specs=pl.BlockSpec((_RB, _K), lambda i, b: (i, 0)),
        scratch_shapes=[pltpu.VMEM((_NCH, _RB, _CB), jnp.float32)],
    )
    return pl.pallas_call(
        _nb_body,
        grid_spec=grid_spec,
        out_shape=jax.ShapeDtypeStruct((_N, _K), jnp.int32),
    )(bounds, g3, ca, cat, cbr, cbt)


def _lnorm(x, s, b):
    mu = jnp.mean(x, axis=1, keepdims=True)
    va = jnp.mean((x - mu) * (x - mu), axis=1, keepdims=True)
    return (x - mu) / jnp.sqrt(va + 1e-5) * s + b


def _lroll(x, s):
    if s == 0:
        return x
    w = x.shape[1]
    s = s % w
    return jnp.concatenate([x[:, w - s:], x[:, :w - s]], axis=1)


def _rot(vec, rc, na):
    w = 3 * na
    lanel = lax.broadcasted_iota(jnp.int32, (1, w), 1) % 3
    out = jnp.zeros_like(vec)
    for s in (-2, -1, 0, 1, 2):
        wgt = None
        for l in range(3):
            j = l - s
            if 0 <= j <= 2:
                col = rc[:, 3 * j + l:3 * j + l + 1]
                wgt = jnp.where(lanel == l, col,
                                jnp.zeros_like(vec) if wgt is None else wgt)
        if wgt is not None:
            out = out + _lroll(vec, s) * wgt
    return out


def _rot_rel(ri, rj):
    lanei = lax.broadcasted_iota(jnp.int32, (1, 9), 1) // 3
    out = jnp.zeros_like(rj)
    for s3 in (-2, -1, 0, 1, 2):
        wgt = None
        for i in range(3):
            j = i - s3
            if 0 <= j <= 2:
                col = ri[:, 3 * j + i:3 * j + i + 1]
                wgt = jnp.where(lanei == i, col,
                                jnp.zeros_like(rj) if wgt is None else wgt)
        if wgt is not None:
            out = out + _lroll(rj, 3 * s3) * wgt
    return out


def _tile3(t3, na):
    w = 3 * na
    lanem = lax.broadcasted_iota(jnp.int32, (1, w), 1) % 3
    return jnp.where(lanem == 0, t3[:, 0:1],
                     jnp.where(lanem == 1, t3[:, 1:2], t3[:, 2:3]))


def _pre_body(f_ref, pos_ref, cb_ref, lns_ref, lnb_ref, wq_ref, wk_ref,
              wv_ref, q_ref, t_ref, r_ref):
    x = _lnorm(f_ref[...], lns_ref[...], lnb_ref[...])
    q_ref[...] = x @ wq_ref[...]
    k = x @ wk_ref[...]
    v = x @ wv_ref[...]
    p42 = pos_ref[...]
    n3 = p42[:, 0:3]; ca = p42[:, 3:6]; c3 = p42[:, 6:9]
    v1 = c3 - ca
    v2 = n3 - ca
    e1 = v1 / jnp.sqrt(jnp.sum(v1 * v1, axis=1, keepdims=True) + 1e-12)
    dot = jnp.sum(e1 * v2, axis=1, keepdims=True)
    u2 = v2 - e1 * dot
    e2 = u2 / jnp.sqrt(jnp.sum(u2 * u2, axis=1, keepdims=True) + 1e-12)
    e1x, e1y, e1z = e1[:, 0:1], e1[:, 1:2], e1[:, 2:3]
    e2x, e2y, e2z = e2[:, 0:1], e2[:, 1:2], e2[:, 2:3]
    e3x = e1y * e2z - e1z * e2y
    e3y = e1z * e2x - e1x * e2z
    e3z = e1x * e2y - e1y * e2x
    r = jnp.concatenate([e1x, e2x, e3x, e1y, e2y, e3y, e1z, e2z, e3z], axis=1)
    r_ref[...] = r
    chf = lax.bitcast_convert_type(cb_ref[:, 0:1], jnp.float32)
    btf = lax.bitcast_convert_type(cb_ref[:, 1:2], jnp.float32)
    pad = jnp.zeros((_RP, _W - 565), jnp.float32)
    t_ref[...] = jnp.concatenate([k, v, p42, r, chf, btf, pad], axis=1)


def _pre_pallas(features, pos42, chain, batch, p):
    cbr = jnp.stack([chain, batch], axis=1)
    return pl.pallas_call(
        _pre_body,
        grid=(_N // _RP,),
        in_specs=[
            pl.BlockSpec((_RP, _D), lambda i: (i, 0)),
            pl.BlockSpec((_RP, 42), lambda i: (i, 0)),
            pl.BlockSpec((_RP, 2), lambda i: (i, 0)),
            pl.BlockSpec((1, _D), lambda i: (0, 0)),
            pl.BlockSpec((1, _D), lambda i: (0, 0)),
            pl.BlockSpec((_D, _D), lambda i: (0, 0)),
            pl.BlockSpec((_D, _D), lambda i: (0, 0)),
            pl.BlockSpec((_D, _D), lambda i: (0, 0)),
        ],
        out_specs=[
            pl.BlockSpec((_RP, _D), lambda i: (i, 0)),
            pl.BlockSpec((_RP, _W), lambda i: (i, 0)),
            pl.BlockSpec((_RP, 9), lambda i: (i, 0)),
        ],
        out_shape=[
            jax.ShapeDtypeStruct((_N, _D), jnp.float32),
            jax.ShapeDtypeStruct((_N, _W), jnp.float32),
            jax.ShapeDtypeStruct((_N, 9), jnp.float32),
        ],
    )(features, pos42, cbr,
      p['ln_attn_s'].reshape(1, _D), p['ln_attn_b'].reshape(1, _D),
      p['wq'], p['wk'], p['wv'])


def _gather_body(tab_ref, idx_ref, out_ref, idx_v, rows_v, sem):
    wid = lax.axis_index("s") * 2 + lax.axis_index("c")
    per = (_N * _K) // _NW
    base = wid * per

    def body(c, carry):
        b = base + c * _CH
        pltpu.sync_copy(idx_ref.at[pl.ds(b, _CH)], idx_v)
        pltpu.async_copy(tab_ref.at[idx_v], rows_v, sem).wait()
        pltpu.sync_copy(rows_v, out_ref.at[pl.ds(b, _CH)])
        return carry

    lax.fori_loop(0, per // _CH, body, 0)


def _gather_sc(tab, idx):
    mesh = plsc.VectorSubcoreMesh(core_axis_name="c", subcore_axis_name="s")
    fn = pl.kernel(
        _gather_body,
        mesh=mesh,
        out_type=jax.ShapeDtypeStruct((_N * _K, _W), jnp.float32),
        scratch_types=[
            pltpu.VMEM((_CH,), jnp.int32),
            pltpu.VMEM((_CH, _W), jnp.float32),
            pltpu.SemaphoreType.DMA,
        ],
    )
    return fn(tab, idx)


def _attn_body(g_ref, nbf_ref, q_ref, f_ref, pos_ref, rtab_ref, cb_ref,
               wrp_ref, wdi_ref, wdr_ref, wro_ref, wpv_ref,
               lps_ref, lpb_ref, mw1_ref, mb1_ref, mw2_ref, mb2_ref,
               wb_ref, wo_ref, bo_ref, lus_ref, lub_ref,
               uw1_ref, ub1_ref, uw2_ref, ub2_ref,
               wu_ref, wg_ref, wout_ref, bout_ref, seg_ref, segt_ref,
               out_ref):
    g = g_ref[...]
    nb = nbf_ref[...]
    posr = pos_ref[...]
    rir = rtab_ref[...]
    cbr = cb_ref[...]

    def expand(a):
        return jnp.broadcast_to(a[:, None, :], (_RA, _K, a.shape[1])
                                ).reshape(_M, a.shape[1])

    i0 = pl.program_id(0) * _RA
    rowid = i0 + lax.broadcasted_iota(jnp.int32, (_RA, 1), 0)
    iie = expand(rowid)
    idxc = jnp.maximum(nb, 0)
    chj = lax.bitcast_convert_type(g[:, _C0C:_C0C + 1], jnp.int32)
    btj = lax.bitcast_convert_type(g[:, _C0B:_C0B + 1], jnp.int32)
    cbi = expand(cbr)
    samec = (chj == cbi[:, 0:1]) & (btj == cbi[:, 1:2])
    rel = jnp.clip(idxc - iie, -32, 32) + 32
    rel = jnp.where(samec, rel, 65)
    oh = (rel == lax.broadcasted_iota(jnp.int32, (1, 66), 1)
          ).astype(jnp.float32)
    feat = oh @ wrp_ref[...]
    posj = g[:, _C0P:_C0P + 42]
    ti3 = expand(posr[:, 3:6])
    caj = posj[:, 3:6]
    dvec = caj - ti3
    d = jnp.sqrt(jnp.sum(dvec * dvec, axis=1, keepdims=True) + 1e-12)
    centers = (lax.broadcasted_iota(jnp.int32, (1, 16), 1)
               .astype(jnp.float32) * (22.0 / 15.0))
    sig = 22.0 / 16.0
    rbf = jnp.exp(-((d - centers) ** 2) / (2 * sig * sig))
    feat = feat + rbf @ wdi_ref[...]
    ri = expand(rir)
    ldir = _rot(dvec, ri, 1)
    ldir = ldir / jnp.sqrt(jnp.sum(ldir * ldir, axis=1, keepdims=True) + 1e-12)
    feat = feat + ldir @ wdr_ref[...]
    rj = g[:, _C0R:_C0R + 9]
    feat = feat + _rot_rel(ri, rj) @ wro_ref[...]
    pv = _rot(posj - _tile3(ti3, _A), ri, _A)
    feat = feat + pv @ wpv_ref[...]
    pair = _lnorm(feat, lps_ref[...], lpb_ref[...])
    h = jax.nn.gelu(pair @ mw1_ref[...] + mb1_ref[...])
    pair = h @ mw2_ref[...] + mb2_ref[...]
    bias = pair @ wb_ref[...]
    kj = g[:, _C0K:_C0K + _D]
    qi = expand(q_ref[...])
    logits = ((qi * kj) @ seg_ref[...]) * (1.0 / jnp.sqrt(jnp.float32(_DK)))
    logits = logits + bias
    pm = (nb != -1).astype(jnp.float32)
    logits = jnp.where(pm > 0, logits, -1e9)
    l3 = logits.reshape(_RA, _K, _H)
    mx = jnp.max(l3, axis=1, keepdims=True)
    e = jnp.exp(l3 - mx)
    attn = e / jnp.sum(e, axis=1, keepdims=True)
    attn = attn * pm.reshape(_RA, _K, 1)
    ae = attn.reshape(_M, _H) @ segt_ref[...]
    vj = g[:, _C0V:_C0V + _D]
    o = jnp.sum((ae * vj).reshape(_RA, _K, _D), axis=1)
    f2 = f_ref[...] + o @ wo_ref[...] + bo_ref[...]
    x2 = _lnorm(f2, lus_ref[...], lub_ref[...])
    lp = _rot(posr - _tile3(posr[:, 3:6], _A), rir, _A)
    hh = jax.nn.gelu(lp @ uw1_ref[...] + ub1_ref[...])
    x2 = x2 + hh @ uw2_ref[...] + ub2_ref[...]
    upd = x2 @ wu_ref[...]
    gate = jax.nn.gelu(x2 @ wg_ref[...])
    out_ref[...] = f2 + (gate * upd) @ wout_ref[...] + bout_ref[...]


def _attn_pallas(gth, nbf, q, features, pos42, rtab, chain, batch, p):
    cbr = jnp.stack([chain, batch], axis=1)
    seg = (lax.broadcasted_iota(jnp.int32, (_D, _H), 0) // _DK
           == lax.broadcasted_iota(jnp.int32, (_D, _H), 1)).astype(jnp.float32)
    segt = seg.T
    r1 = lambda a: a.reshape(1, -1)
    full = lambda shp: pl.BlockSpec(shp, lambda i: (0, 0))
    return pl.pallas_call(
        _attn_body,
        grid=(_N // _RA,),
        in_specs=[
            pl.BlockSpec((_M, _W), lambda i: (i, 0)),
            pl.BlockSpec((_M, 1), lambda i: (i, 0)),
            pl.BlockSpec((_RA, _D), lambda i: (i, 0)),
            pl.BlockSpec((_RA, _D), lambda i: (i, 0)),
            pl.BlockSpec((_RA, 42), lambda i: (i, 0)),
            pl.BlockSpec((_RA, 9), lambda i: (i, 0)),
            pl.BlockSpec((_RA, 2), lambda i: (i, 0)),
            full((66, _P)), full((16, _P)), full((3, _P)), full((9, _P)),
            full((42, _P)),
            full((1, _P)), full((1, _P)),
            full((_P, 2 * _P)), full((1, 2 * _P)),
            full((2 * _P, _P)), full((1, _P)),
            full((_P, _H)), full((_D, _D)), full((1, _D)),
            full((1, _D)), full((1, _D)),
            full((42, 2 * _D)), full((1, 2 * _D)),
            full((2 * _D, _D)), full((1, _D)),
            full((_D, 2 * _D)), full((_D, 2 * _D)),
            full((2 * _D, _D)), full((1, _D)),
            full((_D, _H)), full((_H, _D)),
        ],
        out_specs=pl.BlockSpec((_RA, _D), lambda i: (i, 0)),
        out_shape=jax.ShapeDtypeStruct((_N, _D), jnp.float32),
    )(gth, nbf, q, features, pos42, rtab, cbr,
      p['w_relpos'], p['w_dist'], p['w_dir'], p['w_rot'], p['w_pvec'],
      r1(p['ln_pair_s']), r1(p['ln_pair_b']),
      p['mlp_pair_w1'], r1(p['mlp_pair_b1']),
      p['mlp_pair_w2'], r1(p['mlp_pair_b2']),
      p['wb'], p['wo'], r1(p['bo']),
      r1(p['ln_upd_s']), r1(p['ln_upd_b']),
      p['upd_w1'], r1(p['upd_b1']), p['upd_w2'], r1(p['upd_b2']),
      p['w_update'], p['w_gate'], p['w_out'], r1(p['b_out']),
      seg, segt)


def kernel(features, pos, resi, chain, batch, mask, params):
    p = params
    g = jax.random.gumbel(jax.random.key(42), (_N, _N))
    pos42 = pos.reshape(_N, _A * 3)
    nb = _nb_pallas(pos[:, 1], chain, batch, g)
    q, tab, rtab = _pre_pallas(features, pos42, chain, batch, p)
    idx = jnp.maximum(nb, 0).reshape(-1)
    gth = _gather_sc(tab, idx)
    return _attn_pallas(gth, nb.reshape(-1, 1), q, features, pos42, rtab,
                        chain, batch, p)


# speedup vs baseline: 1.3796x; 1.3796x over previous
import jax
import jax.numpy as jnp
from jax import lax
from jax.experimental import pallas as pl
from jax.experimental.pallas import tpu as pltpu
from jax.experimental.pallas import tpu_sc as plsc

_N = 4096; _D = 256; _P = 64; _H = 8; _DK = _D // _H; _A = 14
_NI = 16; _NS = 16; _K = 64
_RB = 64
_RP = 256
_RA = 64
_M = _RA * _K
_W = 640
_C0K, _C0V, _C0P, _C0R, _C0C, _C0B = 0, 256, 512, 554, 563, 564
_NW, _CH = 32, 128


def _nb_body(g_ref, car_ref, cat_ref, cbr_ref, cbt_ref, nb_ref, vals_ref):
    i0 = pl.program_id(0) * _RB
    ii = i0 + lax.broadcasted_iota(jnp.int32, (_RB, 1), 0)
    jj = lax.broadcasted_iota(jnp.int32, (1, _N), 1)
    xi = car_ref[:, 0:1]; yi = car_ref[:, 1:2]; zi = car_ref[:, 2:3]
    xj = cat_ref[0:1, :]; yj = cat_ref[1:2, :]; zj = cat_ref[2:3, :]
    dx = xi - xj; dy = yi - yj; dz = zi - zj
    d2 = dx * dx + dy * dy + dz * dz
    ci = cbr_ref[:, 0:1]; bi = cbr_ref[:, 1:2]
    cj = cbt_ref[0:1, :]; bj = cbt_ref[1:2, :]
    same_b = bi == bj
    same_c = ci == cj
    within = (jnp.abs(ii - jj) < _NI) & same_b & same_c
    inf = jnp.float32(jnp.inf)
    dm = jnp.where(within | (~same_b), inf, d2)
    vals_ref[...] = dm

    def cbody(t, m):
        v = vals_ref[...]
        m = jnp.min(v, axis=1, keepdims=True)
        vals_ref[...] = jnp.where(v == m, inf, v)
        return m

    cut2 = lax.fori_loop(0, _NS, cbody, jnp.zeros((_RB, 1), jnp.float32))
    within2 = within | (dm < cut2)
    d2e = d2 + 1e-12
    key = jnp.exp(2.0 * g_ref[...]) * (d2e * d2e * d2e)
    key = jnp.where(within2, -1.0, key)
    key = jnp.where(same_b, key, inf)
    vals_ref[...] = key
    kidx = lax.broadcasted_iota(jnp.int32, (1, _K), 1)

    def ebody(t, nb):
        v = vals_ref[...]
        m = jnp.min(v, axis=1, keepdims=True)
        j = jnp.min(jnp.where(v == m, jj, _N), axis=1, keepdims=True)
        vals_ref[...] = jnp.where(jj == j, inf, v)
        col = jnp.where(jnp.isinf(m), -1, j)
        return jnp.where(kidx == t, col, nb)

    nb = lax.fori_loop(0, _K, ebody, jnp.zeros((_RB, _K), jnp.int32))
    nb_ref[...] = nb


def _nb_pallas(ca, chain, batch, g):
    caT = ca.T
    cbr = jnp.stack([chain, batch], axis=1)
    cbt = jnp.stack([chain, batch], axis=0)
    return pl.pallas_call(
        _nb_body,
        grid=(_N // _RB,),
        in_specs=[
            pl.BlockSpec((_RB, _N), lambda i: (i, 0)),
            pl.BlockSpec((_RB, 3), lambda i: (i, 0)),
            pl.BlockSpec((3, _N), lambda i: (0, 0)),
            pl.BlockSpec((_RB, 2), lambda i: (i, 0)),
            pl.BlockSpec((2, _N), lambda i: (0, 0)),
        ],
        out_specs=pl.BlockSpec((_RB, _K), lambda i: (i, 0)),
        out_shape=jax.ShapeDtypeStruct((_N, _K), jnp.int32),
        scratch_shapes=[pltpu.VMEM((_RB, _N), jnp.float32)],
    )(g, ca, caT, cbr, cbt)


def _lnorm(x, s, b):
    mu = jnp.mean(x, axis=1, keepdims=True)
    va = jnp.mean((x - mu) * (x - mu), axis=1, keepdims=True)
    return (x - mu) / jnp.sqrt(va + 1e-5) * s + b


def _lroll(x, s):
    if s == 0:
        return x
    w = x.shape[1]
    s = s % w
    return jnp.concatenate([x[:, w - s:], x[:, :w - s]], axis=1)


def _rot(vec, rc, na):
    w = 3 * na
    lanel = lax.broadcasted_iota(jnp.int32, (1, w), 1) % 3
    out = jnp.zeros_like(vec)
    for s in (-2, -1, 0, 1, 2):
        wgt = None
        for l in range(3):
            j = l - s
            if 0 <= j <= 2:
                col = rc[:, 3 * j + l:3 * j + l + 1]
                wgt = jnp.where(lanel == l, col,
                                jnp.zeros_like(vec) if wgt is None else wgt)
        if wgt is not None:
            out = out + _lroll(vec, s) * wgt
    return out


def _rot_rel(ri, rj):
    lanei = lax.broadcasted_iota(jnp.int32, (1, 9), 1) // 3
    out = jnp.zeros_like(rj)
    for s3 in (-2, -1, 0, 1, 2):
        wgt = None
        for i in range(3):
            j = i - s3
            if 0 <= j <= 2:
                col = ri[:, 3 * j + i:3 * j + i + 1]
                wgt = jnp.where(lanei == i, col,
                                jnp.zeros_like(rj) if wgt is None else wgt)
        if wgt is not None:
            out = out + _lroll(rj, 3 * s3) * wgt
    return out


def _tile3(t3, na):
    w = 3 * na
    lanem = lax.broadcasted_iota(jnp.int32, (1, w), 1) % 3
    return jnp.where(lanem == 0, t3[:, 0:1],
                     jnp.where(lanem == 1, t3[:, 1:2], t3[:, 2:3]))


def _pre_body(f_ref, pos_ref, cb_ref, lns_ref, lnb_ref, wq_ref, wk_ref,
              wv_ref, q_ref, t_ref, r_ref):
    x = _lnorm(f_ref[...], lns_ref[...], lnb_ref[...])
    q_ref[...] = x @ wq_ref[...]
    k = x @ wk_ref[...]
    v = x @ wv_ref[...]
    p42 = pos_ref[...]
    n3 = p42[:, 0:3]; ca = p42[:, 3:6]; c3 = p42[:, 6:9]
    v1 = c3 - ca
    v2 = n3 - ca
    e1 = v1 / jnp.sqrt(jnp.sum(v1 * v1, axis=1, keepdims=True) + 1e-12)
    dot = jnp.sum(e1 * v2, axis=1, keepdims=True)
    u2 = v2 - e1 * dot
    e2 = u2 / jnp.sqrt(jnp.sum(u2 * u2, axis=1, keepdims=True) + 1e-12)
    e1x, e1y, e1z = e1[:, 0:1], e1[:, 1:2], e1[:, 2:3]
    e2x, e2y, e2z = e2[:, 0:1], e2[:, 1:2], e2[:, 2:3]
    e3x = e1y * e2z - e1z * e2y
    e3y = e1z * e2x - e1x * e2z
    e3z = e1x * e2y - e1y * e2x
    r = jnp.concatenate([e1x, e2x, e3x, e1y, e2y, e3y, e1z, e2z, e3z], axis=1)
    r_ref[...] = r
    chf = lax.bitcast_convert_type(cb_ref[:, 0:1], jnp.float32)
    btf = lax.bitcast_convert_type(cb_ref[:, 1:2], jnp.float32)
    pad = jnp.zeros((_RP, _W - 565), jnp.float32)
    t_ref[...] = jnp.concatenate([k, v, p42, r, chf, btf, pad], axis=1)


def _pre_pallas(features, pos42, chain, batch, p):
    cbr = jnp.stack([chain, batch], axis=1)
    return pl.pallas_call(
        _pre_body,
        grid=(_N // _RP,),
        in_specs=[
            pl.BlockSpec((_RP, _D), lambda i: (i, 0)),
            pl.BlockSpec((_RP, 42), lambda i: (i, 0)),
            pl.BlockSpec((_RP, 2), lambda i: (i, 0)),
            pl.BlockSpec((1, _D), lambda i: (0, 0)),
            pl.BlockSpec((1, _D), lambda i: (0, 0)),
            pl.BlockSpec((_D, _D), lambda i: (0, 0)),
            pl.BlockSpec((_D, _D), lambda i: (0, 0)),
            pl.BlockSpec((_D, _D), lambda i: (0, 0)),
        ],
        out_specs=[
            pl.BlockSpec((_RP, _D), lambda i: (i, 0)),
            pl.BlockSpec((_RP, _W), lambda i: (i, 0)),
            pl.BlockSpec((_RP, 9), lambda i: (i, 0)),
        ],
        out_shape=[
            jax.ShapeDtypeStruct((_N, _D), jnp.float32),
            jax.ShapeDtypeStruct((_N, _W), jnp.float32),
            jax.ShapeDtypeStruct((_N, 9), jnp.float32),
        ],
    )(features, pos42, cbr,
      p['ln_attn_s'].reshape(1, _D), p['ln_attn_b'].reshape(1, _D),
      p['wq'], p['wk'], p['wv'])


def _gather_body(tab_ref, idx_ref, out_ref, idx_v, rows_v, sem):
    wid = lax.axis_index("s") * 2 + lax.axis_index("c")
    per = (_N * _K) // _NW
    base = wid * per

    def body(c, carry):
        b = base + c * _CH
        pltpu.sync_copy(idx_ref.at[pl.ds(b, _CH)], idx_v)
        pltpu.async_copy(tab_ref.at[idx_v], rows_v, sem).wait()
        pltpu.sync_copy(rows_v, out_ref.at[pl.ds(b, _CH)])
        return carry

    lax.fori_loop(0, per // _CH, body, 0)


def _gather_sc(tab, idx):
    mesh = plsc.VectorSubcoreMesh(core_axis_name="c", subcore_axis_name="s")
    fn = pl.kernel(
        _gather_body,
        mesh=mesh,
        out_type=jax.ShapeDtypeStruct((_N * _K, _W), jnp.float32),
        scratch_types=[
            pltpu.VMEM((_CH,), jnp.int32),
            pltpu.VMEM((_CH, _W), jnp.float32),
            pltpu.SemaphoreType.DMA,
        ],
    )
    return fn(tab, idx)


def _attn_body(g_ref, nbf_ref, q_ref, f_ref, pos_ref, rtab_ref, cb_ref,
               wrp_ref, wdi_ref, wdr_ref, wro_ref, wpv_ref,
               lps_ref, lpb_ref, mw1_ref, mb1_ref, mw2_ref, mb2_ref,
               wb_ref, wo_ref, bo_ref, lus_ref, lub_ref,
               uw1_ref, ub1_ref, uw2_ref, ub2_ref,
               wu_ref, wg_ref, wout_ref, bout_ref, seg_ref, segt_ref,
               out_ref):
    g = g_ref[...]
    nb = nbf_ref[...]
    posr = pos_ref[...]
    rir = rtab_ref[...]
    cbr = cb_ref[...]

    def expand(a):
        return jnp.broadcast_to(a[:, None, :], (_RA, _K, a.shape[1])
                                ).reshape(_M, a.shape[1])

    i0 = pl.program_id(0) * _RA
    rowid = i0 + lax.broadcasted_iota(jnp.int32, (_RA, 1), 0)
    iie = expand(rowid)
    idxc = jnp.maximum(nb, 0)
    chj = lax.bitcast_convert_type(g[:, _C0C:_C0C + 1], jnp.int32)
    btj = lax.bitcast_convert_type(g[:, _C0B:_C0B + 1], jnp.int32)
    cbi = expand(cbr)
    samec = (chj == cbi[:, 0:1]) & (btj == cbi[:, 1:2])
    rel = jnp.clip(idxc - iie, -32, 32) + 32
    rel = jnp.where(samec, rel, 65)
    oh = (rel == lax.broadcasted_iota(jnp.int32, (1, 66), 1)
          ).astype(jnp.float32)
    feat = oh @ wrp_ref[...]
    posj = g[:, _C0P:_C0P + 42]
    ti3 = expand(posr[:, 3:6])
    caj = posj[:, 3:6]
    dvec = caj - ti3
    d = jnp.sqrt(jnp.sum(dvec * dvec, axis=1, keepdims=True) + 1e-12)
    centers = (lax.broadcasted_iota(jnp.int32, (1, 16), 1)
               .astype(jnp.float32) * (22.0 / 15.0))
    sig = 22.0 / 16.0
    rbf = jnp.exp(-((d - centers) ** 2) / (2 * sig * sig))
    feat = feat + rbf @ wdi_ref[...]
    ri = expand(rir)
    ldir = _rot(dvec, ri, 1)
    ldir = ldir / jnp.sqrt(jnp.sum(ldir * ldir, axis=1, keepdims=True) + 1e-12)
    feat = feat + ldir @ wdr_ref[...]
    rj = g[:, _C0R:_C0R + 9]
    feat = feat + _rot_rel(ri, rj) @ wro_ref[...]
    pv = _rot(posj - _tile3(ti3, _A), ri, _A)
    feat = feat + pv @ wpv_ref[...]
    pair = _lnorm(feat, lps_ref[...], lpb_ref[...])
    h = jax.nn.gelu(pair @ mw1_ref[...] + mb1_ref[...])
    pair = h @ mw2_ref[...] + mb2_ref[...]
    bias = pair @ wb_ref[...]
    kj = g[:, _C0K:_C0K + _D]
    qi = expand(q_ref[...])
    logits = ((qi * kj) @ seg_ref[...]) * (1.0 / jnp.sqrt(jnp.float32(_DK)))
    logits = logits + bias
    pm = (nb != -1).astype(jnp.float32)
    logits = jnp.where(pm > 0, logits, -1e9)
    l3 = logits.reshape(_RA, _K, _H)
    mx = jnp.max(l3, axis=1, keepdims=True)
    e = jnp.exp(l3 - mx)
    attn = e / jnp.sum(e, axis=1, keepdims=True)
    attn = attn * pm.reshape(_RA, _K, 1)
    ae = attn.reshape(_M, _H) @ segt_ref[...]
    vj = g[:, _C0V:_C0V + _D]
    o = jnp.sum((ae * vj).reshape(_RA, _K, _D), axis=1)
    f2 = f_ref[...] + o @ wo_ref[...] + bo_ref[...]
    x2 = _lnorm(f2, lus_ref[...], lub_ref[...])
    lp = _rot(posr - _tile3(posr[:, 3:6], _A), rir, _A)
    hh = jax.nn.gelu(lp @ uw1_ref[...] + ub1_ref[...])
    x2 = x2 + hh @ uw2_ref[...] + ub2_ref[...]
    upd = x2 @ wu_ref[...]
    gate = jax.nn.gelu(x2 @ wg_ref[...])
    out_ref[...] = f2 + (gate * upd) @ wout_ref[...] + bout_ref[...]


def _attn_pallas(gth, nbf, q, features, pos42, rtab, chain, batch, p):
    cbr = jnp.stack([chain, batch], axis=1)
    seg = (lax.broadcasted_iota(jnp.int32, (_D, _H), 0) // _DK
           == lax.broadcasted_iota(jnp.int32, (_D, _H), 1)).astype(jnp.float32)
    segt = seg.T
    r1 = lambda a: a.reshape(1, -1)
    full = lambda shp: pl.BlockSpec(shp, lambda i: (0, 0))
    return pl.pallas_call(
        _attn_body,
        grid=(_N // _RA,),
        in_specs=[
            pl.BlockSpec((_M, _W), lambda i: (i, 0)),
            pl.BlockSpec((_M, 1), lambda i: (i, 0)),
            pl.BlockSpec((_RA, _D), lambda i: (i, 0)),
            pl.BlockSpec((_RA, _D), lambda i: (i, 0)),
            pl.BlockSpec((_RA, 42), lambda i: (i, 0)),
            pl.BlockSpec((_RA, 9), lambda i: (i, 0)),
            pl.BlockSpec((_RA, 2), lambda i: (i, 0)),
            full((66, _P)), full((16, _P)), full((3, _P)), full((9, _P)),
            full((42, _P)),
            full((1, _P)), full((1, _P)),
            full((_P, 2 * _P)), full((1, 2 * _P)),
            full((2 * _P, _P)), full((1, _P)),
            full((_P, _H)), full((_D, _D)), full((1, _D)),
            full((1, _D)), full((1, _D)),
            full((42, 2 * _D)), full((1, 2 * _D)),
            full((2 * _D, _D)), full((1, _D)),
            full((_D, 2 * _D)), full((_D, 2 * _D)),
            full((2 * _D, _D)), full((1, _D)),
            full((_D, _H)), full((_H, _D)),
        ],
        out_specs=pl.BlockSpec((_RA, _D), lambda i: (i, 0)),
        out_shape=jax.ShapeDtypeStruct((_N, _D), jnp.float32),
    )(gth, nbf, q, features, pos42, rtab, cbr,
      p['w_relpos'], p['w_dist'], p['w_dir'], p['w_rot'], p['w_pvec'],
      r1(p['ln_pair_s']), r1(p['ln_pair_b']),
      p['mlp_pair_w1'], r1(p['mlp_pair_b1']),
      p['mlp_pair_w2'], r1(p['mlp_pair_b2']),
      p['wb'], p['wo'], r1(p['bo']),
      r1(p['ln_upd_s']), r1(p['ln_upd_b']),
      p['upd_w1'], r1(p['upd_b1']), p['upd_w2'], r1(p['upd_b2']),
      p['w_update'], p['w_gate'], p['w_out'], r1(p['b_out']),
      seg, segt)


def kernel(features, pos, resi, chain, batch, mask, params):
    p = params
    g = jax.random.gumbel(jax.random.key(42), (_N, _N))
    pos42 = pos.reshape(_N, _A * 3)
    nb = _nb_pallas(pos[:, 1], chain, batch, g)
    q, tab, rtab = _pre_pallas(features, pos42, chain, batch, p)
    idx = jnp.maximum(nb, 0).reshape(-1)
    gth = _gather_sc(tab, idx)
    return _attn_pallas(gth, nb.reshape(-1, 1), q, features, pos42, rtab,
                        chain, batch, p)
